# baseline (speedup 1.0000x reference)
import functools

import jax
import jax.numpy as jnp
from jax import lax
from jax.experimental import pallas as pl
from jax.experimental.pallas import tpu as pltpu
from jax.experimental.pallas import tpu_sc as plsc

HID = 256
B = 256
N = 10000
NPAD = 10240
E = 160000
NBLK = 40
NW = 16
CHUNK = 16
PGROWS = 8
NPAGE = 80
EPW = NPAGE * PGROWS * CHUNK
EPAD = EPW * NW
HALF = NPAD // 2
SLICE = HALF // NW
DUMMY = NPAD - 1


def _sc_agg_body(x_hbm, src_hbm, dst_hbm, z_hbm, src_pg, dst_pg, rows_v, sems, ssem):
    c = lax.axis_index("c")
    s = lax.axis_index("s")
    w = c * NW + s

    pltpu.sync_copy(x_hbm.at[pl.ds(c * HALF + s * SLICE, SLICE)],
                    z_hbm.at[pl.ds(c * HALF + s * SLICE, SLICE)])
    plsc.subcore_barrier()

    def start(pbuf, r, buf):
        pltpu.make_async_copy(x_hbm.at[src_pg.at[pbuf, r]], rows_v.at[buf],
                              sems.at[buf]).start()

    pltpu.sync_copy(src_hbm.at[s, 0], src_pg.at[0])
    pltpu.sync_copy(dst_hbm.at[w, 0], dst_pg.at[0])
    start(0, 0, 0)

    def outer(p, carry):
        pcur = lax.rem(p, 2)
        pnxt = lax.rem(p + 1, 2)

        @pl.when(p < NPAGE - 1)
        def _():
            pltpu.sync_copy(src_hbm.at[s, p + 1], src_pg.at[pnxt])
            pltpu.sync_copy(dst_hbm.at[w, p + 1], dst_pg.at[pnxt])

        for r in range(PGROWS):
            buf = r % 2
            pltpu.make_async_copy(x_hbm.at[src_pg.at[pcur, r]],
                                  rows_v.at[buf], sems.at[buf]).wait()
            if r < PGROWS - 1:
                start(pcur, r + 1, (r + 1) % 2)
            else:
                @pl.when(p < NPAGE - 1)
                def _():
                    start(pnxt, 0, 0)
            pltpu.async_copy(rows_v.at[buf], z_hbm.at[dst_pg.at[pcur, r]], ssem, add=True).wait()
        return carry

    lax.fori_loop(0, NPAGE, outer, 0)


def _sc_aggregate(x, src4, dst4, d):
    mesh = plsc.VectorSubcoreMesh(core_axis_name="c", subcore_axis_name="s")
    return pl.kernel(
        _sc_agg_body,
        mesh=mesh,
        out_type=jax.ShapeDtypeStruct((NPAD, d), jnp.float32),
        scratch_types=[
            pltpu.VMEM((2, PGROWS, CHUNK), jnp.int32),
            pltpu.VMEM((2, PGROWS, CHUNK), jnp.int32),
            pltpu.VMEM((2, CHUNK, d), jnp.float32),
            pltpu.SemaphoreType.DMA((2,)),
            pltpu.SemaphoreType.DMA,
        ],
    )(x, src4, dst4)


def _mm_stats_kernel(z_ref, w1_ref, b1_ref, w2_ref, b2_ref, h_ref, st_ref):
    z = z_ref[...]
    h1 = jnp.maximum(jnp.dot(z, w1_ref[...], preferred_element_type=jnp.float32)
                     + b1_ref[...], 0.0)
    h2 = jnp.maximum(jnp.dot(h1, w2_ref[...], preferred_element_type=jnp.float32)
                     + b2_ref[...], 0.0)
    h_ref[...] = h2
    i = pl.program_id(0)
    rows = lax.broadcasted_iota(jnp.int32, (256, 1), 0) + i * 256
    hm = jnp.where(rows < N, h2, 0.0)

    @pl.when(i == 0)
    def _():
        st_ref[...] = jnp.zeros((8, 256), jnp.float32)

    st_ref[0:1, :] += jnp.sum(hm, axis=0, keepdims=True)
    st_ref[1:2, :] += jnp.sum(hm * hm, axis=0, keepdims=True)


def _mm_stats(z, w1, b1, w2, b2, d):
    return pl.pallas_call(
        _mm_stats_kernel,
        grid=(NBLK,),
        in_specs=[
            pl.BlockSpec((256, d), lambda i: (i, 0)),
            pl.BlockSpec((d, 256), lambda i: (0, 0)),
            pl.BlockSpec((1, 256), lambda i: (0, 0)),
            pl.BlockSpec((256, 256), lambda i: (0, 0)),
            pl.BlockSpec((1, 256), lambda i: (0, 0)),
        ],
        out_specs=[
            pl.BlockSpec((256, 256), lambda i: (i, 0)),
            pl.BlockSpec((8, 256), lambda i: (0, 0)),
        ],
        out_shape=[
            jax.ShapeDtypeStruct((NPAD, 256), jnp.float32),
            jax.ShapeDtypeStruct((8, 256), jnp.float32),
        ],
    )(z, w1, b1[None, :], w2, b2[None, :])


def _scale_shift(st, gamma, beta):
    mu = st[0:1, :] * (1.0 / N)
    var = st[1:2, :] * (1.0 / N)
    scale = gamma * lax.rsqrt(var + 1e-5)
    shift = beta - mu * scale
    return scale, shift


def _stats2_kernel(h_ref, st_ref, o_ref):
    mu = st_ref[0:1, :] * (1.0 / N)
    i = pl.program_id(0)
    rows = lax.broadcasted_iota(jnp.int32, (256, 1), 0) + i * 256
    d = jnp.where(rows < N, h_ref[...] - mu, 0.0)

    @pl.when(i == 0)
    def _():
        o_ref[...] = jnp.zeros((8, 256), jnp.float32)

    o_ref[0:1, :] += st_ref[0:1, :] * (1.0 / NBLK)
    o_ref[1:2, :] += jnp.sum(d * d, axis=0, keepdims=True)


def _stats2(h, st):
    return pl.pallas_call(
        _stats2_kernel,
        grid=(NBLK,),
        in_specs=[
            pl.BlockSpec((256, 256), lambda i: (i, 0)),
            pl.BlockSpec((8, 256), lambda i: (0, 0)),
        ],
        out_specs=pl.BlockSpec((8, 256), lambda i: (0, 0)),
        out_shape=jax.ShapeDtypeStruct((8, 256), jnp.float32),
    )(h, st)


def _norm_kernel(h_ref, st_ref, g_ref, bt_ref, o_ref):
    scale, shift = _scale_shift(st_ref[...], g_ref[...], bt_ref[...])
    o_ref[...] = jnp.maximum(h_ref[...] * scale + shift, 0.0)


def _norm_relu(h, st, gamma, beta):
    return pl.pallas_call(
        _norm_kernel,
        grid=(NBLK,),
        in_specs=[
            pl.BlockSpec((256, 256), lambda i: (i, 0)),
            pl.BlockSpec((8, 256), lambda i: (0, 0)),
            pl.BlockSpec((1, 256), lambda i: (0, 0)),
            pl.BlockSpec((1, 256), lambda i: (0, 0)),
        ],
        out_specs=pl.BlockSpec((256, 256), lambda i: (i, 0)),
        out_shape=jax.ShapeDtypeStruct((NPAD, 256), jnp.float32),
    )(h, st, gamma[None, :], beta[None, :])


def _pool_kernel(h_ref, st_ref, g_ref, bt_ref, bb_ref, pool_ref, cnt_ref):
    scale, shift = _scale_shift(st_ref[...], g_ref[...], bt_ref[...])
    xn = jnp.maximum(h_ref[...] * scale + shift, 0.0)
    bb = jnp.reshape(bb_ref[...], (1, 256))
    ohT = (lax.broadcasted_iota(jnp.int32, (256, 256), 0) == bb).astype(jnp.float32)

    @pl.when(pl.program_id(0) == 0)
    def _():
        pool_ref[...] = jnp.zeros((256, 256), jnp.float32)
        cnt_ref[...] = jnp.zeros((256, 1), jnp.float32)

    pool_ref[...] += jnp.dot(ohT, xn, preferred_element_type=jnp.float32)
    cnt_ref[...] += jnp.reshape(jnp.sum(ohT, axis=1), (256, 1))


def _pool(h, st, gamma, beta, batchf):
    return pl.pallas_call(
        _pool_kernel,
        grid=(NBLK,),
        in_specs=[
            pl.BlockSpec((256, 256), lambda i: (i, 0)),
            pl.BlockSpec((8, 256), lambda i: (0, 0)),
            pl.BlockSpec((1, 256), lambda i: (0, 0)),
            pl.BlockSpec((1, 256), lambda i: (0, 0)),
            pl.BlockSpec((1, 1, 256), lambda i: (i, 0, 0)),
        ],
        out_specs=[
            pl.BlockSpec((256, 256), lambda i: (0, 0)),
            pl.BlockSpec((256, 1), lambda i: (0, 0)),
        ],
        out_shape=[
            jax.ShapeDtypeStruct((256, 256), jnp.float32),
            jax.ShapeDtypeStruct((256, 1), jnp.float32),
        ],
    )(h, st, gamma[None, :], beta[None, :], batchf)


def _head_kernel(lig_pool_ref, lig_cnt_ref, prot_pool_ref, prot_cnt_ref, esm_ref,
                 wv_l_ref, bv_l_ref, wo_l_ref, bo_l_ref,
                 wv_p_ref, bv_p_ref, wo_p_ref, bo_p_ref,
                 ew1_ref, eb1_ref, ew2_ref, eb2_ref,
                 fw1_ref, fb1_ref, fw2_ref, fb2_ref,
                 pw1_ref, pb1_ref, pw2_ref, pb2_ref,
                 out_ref):
    lig_pool = lig_pool_ref[...] / jnp.maximum(lig_cnt_ref[...], 1.0)
    prot_pool = prot_pool_ref[...] / jnp.maximum(prot_cnt_ref[...], 1.0)
    lig_feat = (prot_pool @ wv_l_ref[...] + bv_l_ref[...]) @ wo_l_ref[...] + bo_l_ref[...]
    prot_feat = (lig_pool @ wv_p_ref[...] + bv_p_ref[...]) @ wo_p_ref[...] + bo_p_ref[...]
    esm = jnp.maximum(esm_ref[...] @ ew1_ref[...] + eb1_ref[...], 0.0)
    esm = jnp.maximum(esm @ ew2_ref[...] + eb2_ref[...], 0.0)
    fw1 = fw1_ref[...]
    f = (lig_feat @ fw1[:HID] + prot_feat @ fw1[HID:2 * HID]
         + esm @ fw1[2 * HID:] + fb1_ref[...])
    f = jnp.maximum(f, 0.0)
    f = jnp.maximum(f @ fw2_ref[...] + fb2_ref[...], 0.0)
    h = jnp.maximum(f @ pw1_ref[...] + pb1_ref[...], 0.0)
    out_ref[...] = h @ pw2_ref[...] + pb2_ref[...]


def _head(lig_pool, lig_cnt, prot_pool, prot_cnt, esm_embedding, params):
    al, ap = params["attn_lig"], params["attn_prot"]
    e, fu, pr = params["esm"], params["fusion"], params["pred"]
    args = [lig_pool, lig_cnt, prot_pool, prot_cnt, esm_embedding,
            al["Wv"], al["bv"][None, :], al["Wo"], al["bo"][None, :],
            ap["Wv"], ap["bv"][None, :], ap["Wo"], ap["bo"][None, :],
            e["W1"], e["b1"][None, :], e["W2"], e["b2"][None, :],
            fu["W1"], fu["b1"][None, :], fu["W2"], fu["b2"][None, :],
            pr["W1"], pr["b1"][None, :], pr["W2"], pr["b2"][None, :]]
    return pl.pallas_call(
        _head_kernel,
        out_shape=jax.ShapeDtypeStruct((B, 1), jnp.float32),
    )(*args)


def _prep_edges(edge_index):
    src = jnp.pad(edge_index[0], (0, EPAD - E))
    dst = jnp.pad(edge_index[1], (0, EPAD - E), constant_values=DUMMY)
    src4 = jnp.reshape(src, (NW, NPAGE, PGROWS, CHUNK))
    halves = []
    for c in (0, 1):
        lo = c * HALF
        inh = (dst >= lo) & (dst < lo + HALF) & (dst < N)
        halves.append(jnp.where(inh, dst, DUMMY))
    dst4 = jnp.reshape(jnp.stack(halves), (2 * NW, NPAGE, PGROWS, CHUNK))
    return src4, dst4


def _gin_stack(x, edge_index, layers, batchf):
    src = edge_index[0]
    dst = edge_index[1]
    d = x.shape[1]
    for li, p in enumerate(layers):
        z = x + jnp.zeros((NPAD, d), x.dtype).at[dst].add(x[src])
        w1 = p["W1"]
        if w1.shape[0] != d:
            w1 = jnp.pad(w1, ((0, d - w1.shape[0]), (0, 0)))
        h, st = _mm_stats(z, w1, p["b1"], p["W2"], p["b2"], d)
        st = _stats2(h, st)
        if li < len(layers) - 1:
            x = _norm_relu(h, st, p["gamma"], p["beta"])
            d = 256
        else:
            return _pool(h, st, p["gamma"], p["beta"], batchf)


def kernel(ligand_x, ligand_edge_index, ligand_batch, protein_x, protein_edge_index, protein_batch, esm_embedding, y, params):
    xl = jnp.pad(ligand_x, ((0, NPAD - N), (0, 256 - ligand_x.shape[1])))
    xp = jnp.pad(protein_x, ((0, NPAD - N), (0, 256 - protein_x.shape[1])))
    bl = jnp.pad(ligand_batch, (0, NPAD - N), constant_values=300)
    bp = jnp.pad(protein_batch, (0, NPAD - N), constant_values=300)
    blf = jnp.reshape(bl, (NBLK, 1, 256))
    bpf = jnp.reshape(bp, (NBLK, 1, 256))
    lig_pool, lig_cnt = _gin_stack(xl, ligand_edge_index, params["lig_gin"], blf)
    prot_pool, prot_cnt = _gin_stack(xp, protein_edge_index, params["prot_gin"], bpf)
    return _head(lig_pool, lig_cnt, prot_pool, prot_cnt, esm_embedding, params)
